# baseline (speedup 1.0000x reference)
import functools

import jax
import jax.numpy as jnp
from jax import lax
from jax.experimental import pallas as pl
from jax.experimental.pallas import tpu as pltpu
from jax.experimental.pallas import tpu_sc as plsc

_VOCAB = 100000
_VPAD = 100096
_VT = _VPAD // 128
_EMB = 17
_FIELDS = 26
_FPAD = 28
_ROWS = _FPAD * _VT
_PSTRIDE = _FPAD * _VPAD
_LANES = 16


def _detile(tables):
    t2b = tables.transpose(2, 0, 1)

    def body(in_ref, out_ref):
        x = in_ref[0]
        xp = jnp.concatenate(
            [x, jnp.zeros((_FIELDS, _VPAD - _VOCAB), jnp.float32)], axis=1
        )
        xp = jnp.concatenate(
            [xp, jnp.zeros((_FPAD - _FIELDS, _VPAD), jnp.float32)], axis=0
        )
        out_ref[...] = xp.reshape(_ROWS, 128)

    out = pl.pallas_call(
        body,
        grid=(_EMB,),
        in_specs=[pl.BlockSpec((1, _FIELDS, _VOCAB), lambda e: (e, 0, 0))],
        out_specs=pl.BlockSpec((_ROWS, 128), lambda e: (e, 0)),
        out_shape=jax.ShapeDtypeStruct((_EMB * _ROWS, 128), jnp.float32),
    )(t2b)
    return out.reshape(_EMB * _ROWS * 128)


@functools.lru_cache(maxsize=None)
def _make_gather(B):
    NC, NS = 2, 16
    NW = NC * NS
    rows_per_w = B // NW
    R = 64
    N = R * _FIELDS
    NT = N // _LANES
    n_chunks = rows_per_w // R
    BF = B * _FIELDS

    mesh = plsc.VectorSubcoreMesh(core_axis_name="c", subcore_axis_name="s")

    @functools.partial(
        pl.kernel,
        mesh=mesh,
        out_type=jax.ShapeDtypeStruct((_EMB * BF,), jnp.float32),
        compiler_params=pltpu.CompilerParams(use_tc_tiling_on_sc=False),
        scratch_types=[
            pltpu.VMEM((N,), jnp.int32),
            pltpu.VMEM((N,), jnp.int32),
            pltpu.VMEM((_EMB * N,), jnp.float32),
            pltpu.SemaphoreType.DMA,
        ],
    )
    def k(t1d_hbm, xcat_hbm, out_hbm, xcat_v, idx_v, g_v, sem):
        wid = lax.axis_index("s") * NC + lax.axis_index("c")
        lane = lax.iota(jnp.int32, _LANES)

        def chunk_body(c, _):
            cfb = wid * (rows_per_w * _FIELDS) + c * N
            pltpu.sync_copy(xcat_hbm.at[pl.ds(cfb, N)], xcat_v)

            def idx_body(t, _):
                raw = xcat_v[pl.ds(t * _LANES, _LANES)]
                f = (t * _LANES + lane) % _FIELDS
                idx_v[pl.ds(t * _LANES, _LANES)] = (
                    jnp.clip(raw, 0, _VOCAB - 1) + f * _VPAD
                )
                return 0

            lax.fori_loop(0, NT, idx_body, 0)

            copies = [
                pltpu.async_copy(
                    t1d_hbm.at[pl.ds(ee * _PSTRIDE, _PSTRIDE)].at[idx_v],
                    g_v.at[pl.ds(ee * N, N)],
                    sem,
                )
                for ee in range(_EMB)
            ]
            for cp in copies:
                cp.wait()

            for ee in range(_EMB):
                pltpu.sync_copy(
                    g_v.at[pl.ds(ee * N, N)],
                    out_hbm.at[pl.ds(ee * BF + cfb, N)],
                )
            return 0

        lax.fori_loop(0, n_chunks, chunk_body, 0)

    return k


def kernel(x_num, x_cat, tables):
    B = x_cat.shape[0]
    t1d = _detile(tables)
    xcat_flat = x_cat.astype(jnp.int32).reshape(B * _FIELDS)
    emb_pm = _make_gather(B)(t1d, xcat_flat)
    emb = (
        emb_pm.reshape(_EMB, B * _FIELDS)
        .transpose(1, 0)
        .reshape(B, _FIELDS * _EMB)
    )
    return jnp.concatenate([x_num.astype(jnp.float32), emb], axis=-1)


# speedup vs baseline: 1.0304x; 1.0304x over previous
import functools

import jax
import jax.numpy as jnp
from jax import lax
from jax.experimental import pallas as pl
from jax.experimental.pallas import tpu as pltpu
from jax.experimental.pallas import tpu_sc as plsc

_VOCAB = 100000
_VPAD = 100096
_VT = _VPAD // 128
_EMB = 17
_FIELDS = 26
_FPAD = 28
_ROWS = _FPAD * _VT
_PSTRIDE = _FPAD * _VPAD
_LANES = 16


def _detile(tables):
    t2b = tables.transpose(2, 0, 1)

    def body(in_ref, out_ref):
        x = in_ref[0]
        xp = jnp.concatenate(
            [x, jnp.zeros((_FIELDS, _VPAD - _VOCAB), jnp.float32)], axis=1
        )
        xp = jnp.concatenate(
            [xp, jnp.zeros((_FPAD - _FIELDS, _VPAD), jnp.float32)], axis=0
        )
        out_ref[...] = xp.reshape(_ROWS, 128)

    out = pl.pallas_call(
        body,
        grid=(_EMB,),
        in_specs=[pl.BlockSpec((1, _FIELDS, _VOCAB), lambda e: (e, 0, 0))],
        out_specs=pl.BlockSpec((_ROWS, 128), lambda e: (e, 0)),
        out_shape=jax.ShapeDtypeStruct((_EMB * _ROWS, 128), jnp.float32),
    )(t2b)
    return out.reshape(_EMB * _ROWS * 128)


@functools.lru_cache(maxsize=None)
def _make_gather(B):
    NC, NS = 2, 16
    NW = NC * NS
    rows_per_w = B // NW
    R = 64
    N = R * _FIELDS
    NT = N // _LANES
    n_chunks = rows_per_w // R
    BF = B * _FIELDS

    mesh = plsc.VectorSubcoreMesh(core_axis_name="c", subcore_axis_name="s")

    @functools.partial(
        pl.kernel,
        mesh=mesh,
        out_type=jax.ShapeDtypeStruct((_EMB * BF,), jnp.float32),
        compiler_params=pltpu.CompilerParams(use_tc_tiling_on_sc=False),
        scratch_types=[
            pltpu.VMEM((2, N), jnp.int32),
            pltpu.VMEM((2, N), jnp.int32),
            pltpu.VMEM((2, _EMB * N), jnp.float32),
            pltpu.SemaphoreType.DMA,
            pltpu.SemaphoreType.DMA,
        ],
    )
    def k(t1d_hbm, xcat_hbm, out_hbm, xcat_v, idx_v, g_v, sem0, sem1):
        wid = lax.axis_index("s") * NC + lax.axis_index("c")
        lane = lax.iota(jnp.int32, _LANES)
        sems = (sem0, sem1)
        base = wid * (rows_per_w * _FIELDS)

        def build_fire(c):
            p = c & 1
            cfb = base + c * N
            pltpu.sync_copy(xcat_hbm.at[pl.ds(cfb, N)], xcat_v.at[p])

            def idx_body(t, _):
                raw = xcat_v[p, pl.ds(t * _LANES, _LANES)]
                f = (t * _LANES + lane) % _FIELDS
                idx_v[p, pl.ds(t * _LANES, _LANES)] = (
                    jnp.clip(raw, 0, _VOCAB - 1) + f * _VPAD
                )
                return 0

            lax.fori_loop(0, NT, idx_body, 0)

            return [
                pltpu.async_copy(
                    t1d_hbm.at[pl.ds(ee * _PSTRIDE, _PSTRIDE)].at[idx_v.at[p]],
                    g_v.at[p, pl.ds(ee * N, N)],
                    sems[p],
                )
                for ee in range(_EMB)
            ]

        def drain_write(c, copies):
            p = c & 1
            cfb = base + c * N
            for cp in copies:
                cp.wait()
            for ee in range(_EMB):
                pltpu.sync_copy(
                    g_v.at[p, pl.ds(ee * N, N)],
                    out_hbm.at[pl.ds(ee * BF + cfb, N)],
                )

        inflight = {}
        for c in range(n_chunks):
            if c >= 2:
                drain_write(c - 2, inflight.pop(c - 2))
            inflight[c] = build_fire(c)
        for c in (n_chunks - 2, n_chunks - 1):
            drain_write(c, inflight.pop(c))

    return k


def kernel(x_num, x_cat, tables):
    B = x_cat.shape[0]
    t1d = _detile(tables)
    xcat_flat = x_cat.astype(jnp.int32).reshape(B * _FIELDS)
    emb_pm = _make_gather(B)(t1d, xcat_flat)
    emb = (
        emb_pm.reshape(_EMB, B, _FIELDS)
        .transpose(1, 2, 0)
        .reshape(B, _FIELDS * _EMB)
    )
    return jnp.concatenate([x_num.astype(jnp.float32), emb], axis=-1)


# speedup vs baseline: 1.1554x; 1.1213x over previous
import functools

import jax
import jax.numpy as jnp
from jax import lax
from jax.experimental import pallas as pl
from jax.experimental.pallas import tpu as pltpu
from jax.experimental.pallas import tpu_sc as plsc

_VOCAB = 100000
_VPAD = 100096
_VT = _VPAD // 128
_EMB = 17
_FIELDS = 26
_FPAD = 28
_ROWS = _FPAD * _VT
_PSTRIDE = _FPAD * _VPAD
_LANES = 16


def _detile(tables):
    t2b = tables.transpose(2, 0, 1)

    def body(in_ref, out_ref):
        x = in_ref[0]
        xp = jnp.concatenate(
            [x, jnp.zeros((_FIELDS, _VPAD - _VOCAB), jnp.float32)], axis=1
        )
        xp = jnp.concatenate(
            [xp, jnp.zeros((_FPAD - _FIELDS, _VPAD), jnp.float32)], axis=0
        )
        out_ref[...] = xp.reshape(_ROWS, 128)

    out = pl.pallas_call(
        body,
        grid=(_EMB,),
        in_specs=[pl.BlockSpec((1, _FIELDS, _VOCAB), lambda e: (e, 0, 0))],
        out_specs=pl.BlockSpec((_ROWS, 128), lambda e: (e, 0)),
        out_shape=jax.ShapeDtypeStruct((_EMB * _ROWS, 128), jnp.float32),
    )(t2b)
    return out.reshape(_EMB * _ROWS * 128)


@functools.lru_cache(maxsize=None)
def _make_gather(B):
    NC, NS = 2, 16
    NW = NC * NS
    RW = B // NW
    FPC = 2
    N = FPC * RW
    NT = RW // _LANES
    n_chunks = _FIELDS // FPC

    mesh = plsc.VectorSubcoreMesh(core_axis_name="c", subcore_axis_name="s")

    @functools.partial(
        pl.kernel,
        mesh=mesh,
        out_type=jax.ShapeDtypeStruct((_EMB * _FIELDS * B,), jnp.float32),
        compiler_params=pltpu.CompilerParams(use_tc_tiling_on_sc=False),
        scratch_types=[
            pltpu.VMEM((2, N), jnp.int32),
            pltpu.VMEM((2, N), jnp.int32),
            pltpu.VMEM((2, _EMB * N), jnp.float32),
            pltpu.SemaphoreType.DMA,
            pltpu.SemaphoreType.DMA,
        ],
    )
    def k(t1d_hbm, xcatt_hbm, out_hbm, xcat_v, idx_v, g_v, sem0, sem1):
        wid = lax.axis_index("s") * NC + lax.axis_index("c")
        sems = (sem0, sem1)
        wb = wid * RW

        def build_fire(c):
            p = c & 1
            for fi in range(FPC):
                f = FPC * c + fi
                pltpu.sync_copy(
                    xcatt_hbm.at[pl.ds(f * B + wb, RW)],
                    xcat_v.at[p, pl.ds(fi * RW, RW)],
                )

                def idx_body(t, _):
                    raw = xcat_v[p, pl.ds(fi * RW + t * _LANES, _LANES)]
                    idx_v[p, pl.ds(fi * RW + t * _LANES, _LANES)] = (
                        jnp.clip(raw, 0, _VOCAB - 1) + f * _VPAD
                    )
                    return 0

                lax.fori_loop(0, NT, idx_body, 0)

            return [
                pltpu.async_copy(
                    t1d_hbm.at[pl.ds(ee * _PSTRIDE, _PSTRIDE)].at[idx_v.at[p]],
                    g_v.at[p, pl.ds(ee * N, N)],
                    sems[p],
                )
                for ee in range(_EMB)
            ]

        def drain_write(c, copies):
            p = c & 1
            for cp in copies:
                cp.wait()
            for ee in range(_EMB):
                for fi in range(FPC):
                    f = FPC * c + fi
                    pltpu.sync_copy(
                        g_v.at[p, pl.ds(ee * N + fi * RW, RW)],
                        out_hbm.at[pl.ds((ee * _FIELDS + f) * B + wb, RW)],
                    )

        inflight = {}
        for c in range(n_chunks):
            if c >= 2:
                drain_write(c - 2, inflight.pop(c - 2))
            inflight[c] = build_fire(c)
        for c in (n_chunks - 2, n_chunks - 1):
            drain_write(c, inflight.pop(c))

    return k


def kernel(x_num, x_cat, tables):
    B = x_cat.shape[0]
    t1d = _detile(tables)
    xcatt_flat = x_cat.astype(jnp.int32).T.reshape(_FIELDS * B)
    emb_pm = _make_gather(B)(t1d, xcatt_flat)
    emb = (
        emb_pm.reshape(_EMB, _FIELDS, B)
        .transpose(2, 1, 0)
        .reshape(B, _FIELDS * _EMB)
    )
    return jnp.concatenate([x_num.astype(jnp.float32), emb], axis=-1)
